# baseline (speedup 1.0000x reference)
import jax
import jax.numpy as jnp
from jax import lax
from jax.experimental import pallas as pl
from jax.experimental.pallas import tpu as pltpu
from jax.experimental.pallas import tpu_sc as plsc

_N = 100000
_D = 128


def _scs_body(lam_hbm, table_hbm, out_hbm, lam_s):
    pltpu.sync_copy(lam_hbm, lam_s)
    lam = lam_s[0]
    x = lam * jnp.float32(_N)
    idx = x.astype(jnp.int32)
    idx = jnp.where(idx.astype(jnp.float32) > x, idx - 1, idx)
    idx = jnp.clip(idx, 0, _N - 1)
    pltpu.sync_copy(table_hbm.at[idx], out_hbm)


def kernel(lambd, intervals):
    lam1 = jnp.asarray(lambd, jnp.float32).reshape((1,))
    mesh = plsc.ScalarSubcoreMesh(axis_name="c", num_cores=1)
    run = pl.kernel(
        _scs_body,
        mesh=mesh,
        out_type=jax.ShapeDtypeStruct((_D,), jnp.float32),
        scratch_types=[
            pltpu.SMEM((1,), jnp.float32),
        ],
        compiler_params=pltpu.CompilerParams(skip_device_barrier=True),
    )
    return run(lam1, intervals)
